# Initial kernel scaffold; baseline (speedup 1.0000x reference)
import jax
import jax.numpy as jnp
from jax.experimental import pallas as pl


def kernel(probas, labels):
    raise NotImplementedError("write your pallas kernel here")


# speedup vs baseline: 63.3670x; 63.3670x over previous
import functools

import jax
import jax.numpy as jnp
from jax import lax
from jax.experimental import pallas as pl
from jax.experimental.pallas import tpu as pltpu
from jax.experimental.pallas import tpu_sc as plsc

NB = 2048
NCLS = 10
HIST = 2 * NB
C_FULL = 19

N_PIX = 8 * 512 * 512
NW = 32
PIX_PER_W = N_PIX // NW
CHUNK = 16384
N_SUB = PIX_PER_W // CHUNK


def _keys_body(pro_ref, lab_ref, out_ref):
    x = pro_ref[0]
    lab = lab_ref[0]
    mx = jnp.max(x, axis=0)
    ex = jnp.exp(x - mx[None])
    den = jnp.sum(ex, axis=0)
    inv_den = 1.0 / den
    for c in range(NCLS):
        p = ex[c] * inv_den
        fg = lab == c
        e = jnp.where(fg, 1.0 - p, p)
        b = jnp.minimum((e * NB).astype(jnp.int32), NB - 1)
        key = b + jnp.where(fg, NB, 0)
        out_ref[c, 0] = key


def _compute_keys(probas, labels):
    B, C, H, W = probas.shape
    BH = 128
    grid = (B, H // BH)
    return pl.pallas_call(
        _keys_body,
        grid=grid,
        in_specs=[
            pl.BlockSpec((1, C, BH, W), lambda b, h: (b, 0, h, 0)),
            pl.BlockSpec((1, BH, W), lambda b, h: (b, h, 0)),
        ],
        out_specs=pl.BlockSpec((NCLS, 1, BH, W), lambda b, h: (0, b, h, 0)),
        out_shape=jax.ShapeDtypeStruct((NCLS, B, H, W), jnp.int32),
    )(probas, labels)


def _hist_body(keys_hbm, out_hbm, hist_v, buf_v):
    wid = lax.axis_index("s") * 2 + lax.axis_index("c")
    base = wid * PIX_PER_W

    def zero(i, _):
        hist_v[pl.ds(i * 16, 16)] = jnp.zeros((16,), jnp.int32)
        return 0

    lax.fori_loop(0, (NCLS * HIST) // 16, zero, 0)

    ones = jnp.ones((16,), jnp.int32)
    for c in range(NCLS):
        off = c * HIST
        for sub in range(N_SUB):
            pltpu.sync_copy(
                keys_hbm.at[pl.ds(c * N_PIX + base + sub * CHUNK, CHUNK)],
                buf_v)

            def scat(i, _):
                k = buf_v[pl.ds(i * 16, 16)]
                plsc.addupdate_scatter(hist_v, [k + off], ones)
                return 0

            lax.fori_loop(0, CHUNK // 16, scat, 0)

    pltpu.sync_copy(hist_v, out_hbm.at[pl.ds(wid * (NCLS * HIST), NCLS * HIST)])


def _histogram(keys_flat):
    mesh = plsc.VectorSubcoreMesh(core_axis_name="c", subcore_axis_name="s")
    k = functools.partial(
        pl.kernel,
        mesh=mesh,
        out_type=jax.ShapeDtypeStruct((NW * NCLS * HIST,), jnp.int32),
        scratch_types=[
            pltpu.VMEM((NCLS * HIST,), jnp.int32),
            pltpu.VMEM((CHUNK,), jnp.int32),
        ],
        compiler_params=pltpu.CompilerParams(needs_layout_passes=False),
    )(_hist_body)
    return k(keys_flat)


def _finish_body(part_hbm, out_hbm, acc_v, buf_v, res_v):
    wid = lax.axis_index("s") * 2 + lax.axis_index("c")

    @pl.when(wid < NCLS)
    def _():
        c = wid

        def zero(i, _):
            acc_v[pl.ds(i * 16, 16)] = jnp.zeros((16,), jnp.int32)
            return 0

        lax.fori_loop(0, HIST // 16, zero, 0)

        def add_tile(t, _):
            pltpu.sync_copy(part_hbm.at[pl.ds(t * (NCLS * HIST) + c * HIST,
                                              HIST)], buf_v)

            def add(i, _):
                s = pl.ds(i * 16, 16)
                acc_v[s] = acc_v[s] + buf_v[s]
                return 0

            lax.fori_loop(0, HIST // 16, add, 0)
            return 0

        lax.fori_loop(0, NW, add_tile, 0)

        def gsum(i, gv):
            return gv + acc_v[pl.ds(NB + i * 16, 16)].astype(jnp.float32)

        gvec = lax.fori_loop(0, NB // 16, gsum, jnp.zeros((16,), jnp.float32))
        G = jnp.sum(gvec)

        iota = lax.iota(jnp.int32, 16)

        def step(j, carry):
            cn, cf, cv = carry
            v = NB // 16 - 1 - j
            hn = acc_v[pl.ds(v * 16, 16)].astype(jnp.float32)
            hf = acc_v[pl.ds(NB + v * 16, 16)].astype(jnp.float32)
            tn = jnp.sum(hn)
            tf = jnp.sum(hf)
            sn = (tn - plsc.cumsum(hn)) + hn + cn
            sf = (tf - plsc.cumsum(hf)) + hf + cf
            n = sn + sf
            denom = jnp.maximum(G + n - sf, 1.0)
            term = n / denom
            coef = jnp.where((iota == 0) & (v == 0), 0.5, 1.0)
            return cn + tn, cf + tf, cv + term * coef

        _, _, cvec = lax.fori_loop(
            0, NB // 16, step,
            (jnp.float32(0), jnp.float32(0), jnp.zeros((16,), jnp.float32)))

        contrib = jnp.sum(cvec) * jnp.float32(1.0 / NB)
        contrib = jnp.where(G > 0, contrib, jnp.float32(0))
        res_v[...] = jnp.broadcast_to(contrib, (16,))
        pltpu.sync_copy(res_v, out_hbm.at[pl.ds(c * 16, 16)])


def _finish(partials):
    mesh = plsc.VectorSubcoreMesh(core_axis_name="c", subcore_axis_name="s")
    k = functools.partial(
        pl.kernel,
        mesh=mesh,
        out_type=jax.ShapeDtypeStruct((NCLS * 16,), jnp.float32),
        scratch_types=[
            pltpu.VMEM((HIST,), jnp.int32),
            pltpu.VMEM((HIST,), jnp.int32),
            pltpu.VMEM((16,), jnp.float32),
        ],
        compiler_params=pltpu.CompilerParams(needs_layout_passes=False),
    )(_finish_body)
    return k(partials)


def kernel(probas, labels):
    keys = _compute_keys(probas, labels)
    partials = _histogram(keys.reshape(-1))
    res = _finish(partials)
    return jnp.sum(res.reshape(NCLS, 16)[:, 0]) * jnp.float32(1.0 / NCLS)


# speedup vs baseline: 251.4170x; 3.9676x over previous
import functools

import jax
import jax.numpy as jnp
from jax import lax
from jax.experimental import pallas as pl
from jax.experimental.pallas import tpu as pltpu
from jax.experimental.pallas import tpu_sc as plsc

NB = 2048
NCLS = 10
HIST = 2 * NB
C_FULL = 19

N_PIX = 8 * 512 * 512
NW = 32
PIX_PER_W = N_PIX // NW
CHUNK = 16384
N_SUB = PIX_PER_W // 2 // CHUNK


def _keys_body(pro_ref, lab_ref, out_ref):
    x = pro_ref[0]
    lab = lab_ref[0]
    mx = jnp.max(x, axis=0)
    ex = jnp.exp(x - mx[None])
    den = jnp.sum(ex, axis=0)
    inv_den = 1.0 / den
    for c in range(NCLS):
        p = ex[c] * inv_den
        fg = lab == c
        e = jnp.where(fg, 1.0 - p, p)
        b = jnp.minimum((e * NB).astype(jnp.int32), NB - 1)
        key = b + jnp.where(fg, NB, 0)
        out_ref[c, 0] = key[:64] | (key[64:] << 16)


def _compute_keys(probas, labels, b0, nb):
    _, C, H, W = probas.shape
    BH = 128
    grid = (nb, H // BH)
    return pl.pallas_call(
        _keys_body,
        grid=grid,
        in_specs=[
            pl.BlockSpec((1, C, BH, W), lambda b, h: (b + b0, 0, h, 0)),
            pl.BlockSpec((1, BH, W), lambda b, h: (b + b0, h, 0)),
        ],
        out_specs=pl.BlockSpec((NCLS, 1, BH // 2, W), lambda b, h: (0, b, h, 0)),
        out_shape=jax.ShapeDtypeStruct((NCLS, nb, H // 2, W), jnp.int32),
    )(probas, labels)


def _histogram(keys2):
    rows_total = keys2.shape[0]
    rows_cls = rows_total // NCLS
    rows_w = rows_cls // NW
    crows = min(CHUNK // 512, rows_w)
    n_sub = rows_w // crows

    def _hist_body(keys_hbm, out_hbm, hist_v, buf0_v, buf1_v, red0_v, red1_v,
                   shist_s, sem0, sem1):
        cid = lax.axis_index("c")
        sid = lax.axis_index("s")
        wid = sid * 2 + cid

        @plsc.parallel_loop(0, (NCLS * HIST) // 16, unroll=8)
        def _(i):
            hist_v[pl.ds(i * 16, 16)] = jnp.zeros((16,), jnp.int32)

        ones = jnp.ones((16,), jnp.int32)
        chunks = [(c, sub) for c in range(NCLS) for sub in range(n_sub)]
        bufs = (buf0_v, buf1_v)
        sems = (sem0, sem1)

        def start(t):
            c, sub = chunks[t]
            row0 = c * rows_cls + wid * rows_w + sub * crows
            return pltpu.async_copy(keys_hbm.at[pl.ds(row0, crows)],
                                    bufs[t % 2], sems[t % 2])

        handles = [start(0), None]
        for t in range(len(chunks)):
            if t + 1 < len(chunks):
                handles[(t + 1) % 2] = start(t + 1)
            handles[t % 2].wait()
            buf = bufs[t % 2]
            hsub = hist_v.at[pl.ds(chunks[t][0] * HIST, HIST)]

            @plsc.parallel_loop(0, crows)
            def _(r, buf=buf, hsub=hsub):
                @plsc.parallel_loop(0, 512 // 16, unroll=8)
                def _(j):
                    v = buf[r, pl.ds(j * 16, 16)]
                    plsc.addupdate_scatter(hsub, [v & 0xFFFF], ones)
                    plsc.addupdate_scatter(
                        hsub, [lax.shift_right_logical(v, 16)], ones)

        HL = NCLS * HIST
        STRIPE = HL // 16
        pltpu.sync_copy(hist_v, shist_s.at[pl.ds(sid * HL, HL)])
        plsc.subcore_barrier()

        sbase = sid * STRIPE
        pltpu.sync_copy(shist_s.at[pl.ds(sbase, STRIPE)], red0_v)
        for t in range(1, 16):
            pltpu.sync_copy(shist_s.at[pl.ds(t * HL + sbase, STRIPE)], red1_v)

            @plsc.parallel_loop(0, STRIPE // 16, unroll=8)
            def _(i):
                s = pl.ds(i * 16, 16)
                red0_v[s] = red0_v[s] + red1_v[s]

        pltpu.sync_copy(red0_v, out_hbm.at[pl.ds(cid * HL + sbase, STRIPE)])

    mesh = plsc.VectorSubcoreMesh(core_axis_name="c", subcore_axis_name="s")
    k = functools.partial(
        pl.kernel,
        mesh=mesh,
        out_type=jax.ShapeDtypeStruct((2 * NCLS * HIST,), jnp.int32),
        name="lovasz_hist",
        scratch_types=[
            pltpu.VMEM((NCLS * HIST,), jnp.int32),
            pltpu.VMEM((crows, 512), jnp.int32),
            pltpu.VMEM((crows, 512), jnp.int32),
            pltpu.VMEM((NCLS * HIST // 16,), jnp.int32),
            pltpu.VMEM((NCLS * HIST // 16,), jnp.int32),
            pltpu.VMEM_SHARED((16 * NCLS * HIST,), jnp.int32),
            pltpu.SemaphoreType.DMA,
            pltpu.SemaphoreType.DMA,
        ],
        compiler_params=pltpu.CompilerParams(needs_layout_passes=False),
    )(_hist_body)
    return k(keys2)


def _finish_body(part0_hbm, part1_hbm, out_hbm, acc_v, buf_v, res_v):
    wid = lax.axis_index("s") * 2 + lax.axis_index("c")

    @pl.when(wid < NCLS)
    def _():
        c = wid

        pltpu.sync_copy(part0_hbm.at[pl.ds(c * HIST, HIST)], acc_v)
        for part, half in ((part0_hbm, 1), (part1_hbm, 0), (part1_hbm, 1)):
            pltpu.sync_copy(
                part.at[pl.ds(half * NCLS * HIST + c * HIST, HIST)], buf_v)

            @plsc.parallel_loop(0, HIST // 16, unroll=8)
            def _(i):
                s = pl.ds(i * 16, 16)
                acc_v[s] = acc_v[s] + buf_v[s]

        def gsum(i, gv):
            return gv + acc_v[pl.ds(NB + i * 16, 16)].astype(jnp.float32)

        gvec = lax.fori_loop(0, NB // 16, gsum, jnp.zeros((16,), jnp.float32))
        G = jnp.sum(gvec)

        iota = lax.iota(jnp.int32, 16)

        def step(j, carry):
            cn, cf, cv = carry
            v = NB // 16 - 1 - j
            hn = acc_v[pl.ds(v * 16, 16)].astype(jnp.float32)
            hf = acc_v[pl.ds(NB + v * 16, 16)].astype(jnp.float32)
            tn = jnp.sum(hn)
            tf = jnp.sum(hf)
            sn = (tn - plsc.cumsum(hn)) + hn + cn
            sf = (tf - plsc.cumsum(hf)) + hf + cf
            n = sn + sf
            denom = jnp.maximum(G + n - sf, 1.0)
            term = n / denom
            coef = jnp.where((iota == 0) & (v == 0), 0.5, 1.0)
            return cn + tn, cf + tf, cv + term * coef

        _, _, cvec = lax.fori_loop(
            0, NB // 16, step,
            (jnp.float32(0), jnp.float32(0), jnp.zeros((16,), jnp.float32)))

        contrib = jnp.sum(cvec) * jnp.float32(1.0 / NB)
        contrib = jnp.where(G > 0, contrib, jnp.float32(0))
        res_v[...] = jnp.broadcast_to(contrib, (16,))
        pltpu.sync_copy(res_v, out_hbm.at[pl.ds(c * 16, 16)])


def _finish(part0, part1):
    mesh = plsc.VectorSubcoreMesh(core_axis_name="c", subcore_axis_name="s")
    k = functools.partial(
        pl.kernel,
        mesh=mesh,
        out_type=jax.ShapeDtypeStruct((NCLS * 16,), jnp.float32),
        name="lovasz_finish",
        scratch_types=[
            pltpu.VMEM((HIST,), jnp.int32),
            pltpu.VMEM((HIST,), jnp.int32),
            pltpu.VMEM((16,), jnp.float32),
        ],
        compiler_params=pltpu.CompilerParams(needs_layout_passes=False),
    )(_finish_body)
    return k(part0, part1)


def kernel(probas, labels):
    keys0 = _compute_keys(probas, labels, 0, 4)
    part0 = _histogram(keys0.reshape(NCLS * 4 * 256, 512))
    keys1 = _compute_keys(probas, labels, 4, 4)
    part1 = _histogram(keys1.reshape(NCLS * 4 * 256, 512))
    res = _finish(part0, part1)
    return jnp.sum(res.reshape(NCLS, 16)[:, 0]) * jnp.float32(1.0 / NCLS)
